# baseline (speedup 1.0000x reference)
import functools

import jax
import jax.numpy as jnp
from jax import lax
from jax.experimental import pallas as pl
from jax.experimental.pallas import tpu as pltpu
from jax.experimental.pallas import tpu_sc as plsc

P = 0.05
_LANES = 16
_NCORES = 2
_NSUB = 16
_NTILES = _NCORES * _NSUB


def _drop16(n):
    k1 = jnp.uint32(0)
    k2 = jnp.uint32(42)
    ks0, ks1, ks2 = k1, k2, k1 ^ k2 ^ jnp.uint32(0x1BD11BDA)
    rots = ((13, 15, 26, 6), (17, 29, 16, 24))
    kseq = ((ks1, ks2), (ks2, ks0), (ks0, ks1), (ks1, ks2), (ks2, ks0))
    x0 = jnp.zeros((16,), jnp.uint32) + ks0
    x1 = n + ks1
    for i in range(5):
        for r in rots[i % 2]:
            x0 = x0 + x1
            x1 = (x1 << jnp.uint32(r)) | (x1 >> jnp.uint32(32 - r))
            x1 = x0 ^ x1
        ka, kb = kseq[i]
        x0 = x0 + ka
        x1 = x1 + kb + jnp.uint32(i + 1)
    bits = x0 ^ x1
    return (bits >> jnp.uint32(9)) < jnp.uint32(419431)


@functools.partial(jax.jit, static_argnames=("n", "tpw"))
def _node_drop_masks(m, *, n, tpw):
    mesh = plsc.VectorSubcoreMesh(
        core_axis_name="c", subcore_axis_name="s", num_cores=_NCORES
    )

    @functools.partial(
        pl.kernel,
        mesh=mesh,
        out_type=jax.ShapeDtypeStruct((3 * n,), jnp.int32),
        scratch_types=[pltpu.VMEM((3 * tpw,), jnp.int32)],
    )
    def body(m_hbm, o_hbm, buf):
        wid = lax.axis_index("s") * _NCORES + lax.axis_index("c")
        base = pl.multiple_of(jnp.minimum(wid * tpw, n - tpw), 8)
        for k in range(3):
            pltpu.sync_copy(
                m_hbm.at[pl.ds(k * n + base, tpw)], buf.at[pl.ds(k * tpw, tpw)]
            )
        zero = jnp.zeros((16,), jnp.int32)

        def chunk(c, carry):
            off = c * _LANES
            nvec = (base + off).astype(jnp.uint32) + lax.iota(jnp.uint32, 16)
            drop = _drop16(nvec)
            for k in range(3):
                sl = pl.ds(k * tpw + off, _LANES)
                buf[sl] = jnp.where(drop, zero, buf[sl])
            return carry

        lax.fori_loop(0, tpw // _LANES, chunk, 0)
        for k in range(3):
            pltpu.sync_copy(
                buf.at[pl.ds(k * tpw, tpw)], o_hbm.at[pl.ds(k * n + base, tpw)]
            )

    return body(m)


def kernel(x, edge_index, y, train_mask, test_mask, val_mask):
    n = train_mask.shape[0]
    chunk = _NTILES * _LANES
    tpw = (-(-n // chunk)) * _LANES
    m = jnp.concatenate([
        train_mask.astype(jnp.int32),
        test_mask.astype(jnp.int32),
        val_mask.astype(jnp.int32),
    ])
    out = _node_drop_masks(m, n=n, tpw=tpw).astype(jnp.bool_)
    return (x, edge_index, y, out[0:n], out[2 * n:3 * n], out[n:2 * n])


# speedup vs baseline: 1.0233x; 1.0233x over previous
import functools

import jax
import jax.numpy as jnp
from jax import lax
from jax.experimental import pallas as pl
from jax.experimental.pallas import tpu as pltpu
from jax.experimental.pallas import tpu_sc as plsc

P = 0.05
_LANES = 16
_NCORES = 1
_NSUB = 16
_NTILES = _NCORES * _NSUB


def _drop16(n):
    k1 = jnp.uint32(0)
    k2 = jnp.uint32(42)
    ks0, ks1, ks2 = k1, k2, k1 ^ k2 ^ jnp.uint32(0x1BD11BDA)
    rots = ((13, 15, 26, 6), (17, 29, 16, 24))
    kseq = ((ks1, ks2), (ks2, ks0), (ks0, ks1), (ks1, ks2), (ks2, ks0))
    x0 = jnp.zeros((16,), jnp.uint32) + ks0
    x1 = n + ks1
    for i in range(5):
        for r in rots[i % 2]:
            x0 = x0 + x1
            x1 = (x1 << jnp.uint32(r)) | (x1 >> jnp.uint32(32 - r))
            x1 = x0 ^ x1
        ka, kb = kseq[i]
        x0 = x0 + ka
        x1 = x1 + kb + jnp.uint32(i + 1)
    bits = x0 ^ x1
    return (bits >> jnp.uint32(9)) < jnp.uint32(419431)


@functools.partial(jax.jit, static_argnames=("n", "tpw"))
def _node_drop_masks(m, *, n, tpw):
    mesh = plsc.VectorSubcoreMesh(
        core_axis_name="c", subcore_axis_name="s", num_cores=_NCORES
    )

    @functools.partial(
        pl.kernel,
        mesh=mesh,
        out_type=jax.ShapeDtypeStruct((3 * n,), jnp.int32),
        scratch_types=[pltpu.VMEM((3 * tpw,), jnp.int32)],
    )
    def body(m_hbm, o_hbm, buf):
        wid = lax.axis_index("s") * _NCORES + lax.axis_index("c")
        base = pl.multiple_of(jnp.minimum(wid * tpw, n - tpw), 8)
        for k in range(3):
            pltpu.sync_copy(
                m_hbm.at[pl.ds(k * n + base, tpw)], buf.at[pl.ds(k * tpw, tpw)]
            )
        zero = jnp.zeros((16,), jnp.int32)

        def chunk(c, carry):
            off = c * _LANES
            nvec = (base + off).astype(jnp.uint32) + lax.iota(jnp.uint32, 16)
            drop = _drop16(nvec)
            for k in range(3):
                sl = pl.ds(k * tpw + off, _LANES)
                buf[sl] = jnp.where(drop, zero, buf[sl])
            return carry

        lax.fori_loop(0, tpw // _LANES, chunk, 0)
        for k in range(3):
            pltpu.sync_copy(
                buf.at[pl.ds(k * tpw, tpw)], o_hbm.at[pl.ds(k * n + base, tpw)]
            )

    return body(m)


def kernel(x, edge_index, y, train_mask, test_mask, val_mask):
    n = train_mask.shape[0]
    chunk = _NTILES * _LANES
    tpw = (-(-n // chunk)) * _LANES
    m = jnp.concatenate([
        train_mask.astype(jnp.int32),
        test_mask.astype(jnp.int32),
        val_mask.astype(jnp.int32),
    ])
    out = _node_drop_masks(m, n=n, tpw=tpw).astype(jnp.bool_)
    return (x, edge_index, y, out[0:n], out[2 * n:3 * n], out[n:2 * n])


# speedup vs baseline: 1.1341x; 1.1082x over previous
import functools

import jax
import jax.numpy as jnp
from jax import lax
from jax.experimental import pallas as pl
from jax.experimental.pallas import tpu as pltpu
from jax.experimental.pallas import tpu_sc as plsc

P = 0.05
_LANES = 16
_NCORES = 1
_NSUB = 16
_NTILES = _NCORES * _NSUB


def _drop16(n):
    k1 = jnp.uint32(0)
    k2 = jnp.uint32(42)
    ks0, ks1, ks2 = k1, k2, k1 ^ k2 ^ jnp.uint32(0x1BD11BDA)
    rots = ((13, 15, 26, 6), (17, 29, 16, 24))
    kseq = ((ks1, ks2), (ks2, ks0), (ks0, ks1), (ks1, ks2), (ks2, ks0))
    x0 = jnp.zeros((16,), jnp.uint32) + ks0
    x1 = n + ks1
    for i in range(5):
        for r in rots[i % 2]:
            x0 = x0 + x1
            x1 = (x1 << jnp.uint32(r)) | (x1 >> jnp.uint32(32 - r))
            x1 = x0 ^ x1
        ka, kb = kseq[i]
        x0 = x0 + ka
        x1 = x1 + kb + jnp.uint32(i + 1)
    bits = x0 ^ x1
    return (bits >> jnp.uint32(9)) < jnp.uint32(419431)


@functools.partial(jax.jit, static_argnames=("n", "tpw"))
def _node_drop_masks(m, *, n, tpw):
    mesh = plsc.VectorSubcoreMesh(
        core_axis_name="c", subcore_axis_name="s", num_cores=_NCORES
    )

    @functools.partial(
        pl.kernel,
        mesh=mesh,
        out_type=jax.ShapeDtypeStruct((3 * n,), jnp.int32),
        scratch_types=[pltpu.VMEM((3 * tpw,), jnp.int32)],
    )
    def body(m_hbm, o_hbm, buf):
        wid = lax.axis_index("s") * _NCORES + lax.axis_index("c")
        base = pl.multiple_of(jnp.minimum(wid * tpw, n - tpw), 8)
        for k in range(3):
            pltpu.sync_copy(
                m_hbm.at[pl.ds(k * n + base, tpw)], buf.at[pl.ds(k * tpw, tpw)]
            )
        zero = jnp.zeros((16,), jnp.int32)

        def chunk(c, carry):
            off = c * _LANES
            nvec = (base + off).astype(jnp.uint32) + lax.iota(jnp.uint32, 16)
            drop = _drop16(nvec)
            for k in range(3):
                sl = pl.ds(k * tpw + off, _LANES)
                buf[sl] = jnp.where(drop, zero, buf[sl])
            return carry

        lax.fori_loop(0, tpw // _LANES, chunk, 0)
        for k in range(3):
            pltpu.sync_copy(
                buf.at[pl.ds(k * tpw, tpw)], o_hbm.at[pl.ds(k * n + base, tpw)]
            )

    return body(m)


def kernel(x, edge_index, y, train_mask, test_mask, val_mask):
    n = train_mask.shape[0]
    chunk = _NTILES * _LANES
    tpw = (-(-n // chunk)) * _LANES
    m = jnp.concatenate([
        train_mask.astype(jnp.int32),
        test_mask.astype(jnp.int32),
        val_mask.astype(jnp.int32),
    ])
    out = _node_drop_masks(m, n=n, tpw=tpw).astype(jnp.bool_)
    z = lax.optimization_barrier(jnp.int32(0))
    x_out = lax.bitcast_convert_type(
        lax.bitcast_convert_type(x, jnp.int32) ^ z, jnp.float32
    )
    e_out = edge_index ^ z.astype(edge_index.dtype)
    y_out = y ^ z.astype(y.dtype)
    return (x_out, e_out, y_out, out[0:n], out[2 * n:3 * n], out[n:2 * n])


# speedup vs baseline: 1.1762x; 1.0371x over previous
import functools

import jax
import jax.numpy as jnp
from jax import lax
from jax.experimental import pallas as pl
from jax.experimental.pallas import tpu as pltpu
from jax.experimental.pallas import tpu_sc as plsc

P = 0.05
_LANES = 16
_NCORES = 1
_NSUB = 16
_NTILES = _NCORES * _NSUB


def _drop16(n):
    k1 = jnp.uint32(0)
    k2 = jnp.uint32(42)
    ks0, ks1, ks2 = k1, k2, k1 ^ k2 ^ jnp.uint32(0x1BD11BDA)
    rots = ((13, 15, 26, 6), (17, 29, 16, 24))
    kseq = ((ks1, ks2), (ks2, ks0), (ks0, ks1), (ks1, ks2), (ks2, ks0))
    x0 = jnp.zeros((16,), jnp.uint32) + ks0
    x1 = n + ks1
    for i in range(5):
        for r in rots[i % 2]:
            x0 = x0 + x1
            x1 = (x1 << jnp.uint32(r)) | (x1 >> jnp.uint32(32 - r))
            x1 = x0 ^ x1
        ka, kb = kseq[i]
        x0 = x0 + ka
        x1 = x1 + kb + jnp.uint32(i + 1)
    bits = x0 ^ x1
    return (bits >> jnp.uint32(9)) < jnp.uint32(419431)


@functools.partial(jax.jit, static_argnames=("n", "tpw"))
def _node_drop_masks(m, *, n, tpw):
    mesh = plsc.VectorSubcoreMesh(
        core_axis_name="c", subcore_axis_name="s", num_cores=_NCORES
    )

    @functools.partial(
        pl.kernel,
        mesh=mesh,
        out_type=jax.ShapeDtypeStruct((3 * n,), jnp.int32),
        scratch_types=[pltpu.VMEM((3 * tpw,), jnp.int32)],
    )
    def body(m_hbm, o_hbm, buf):
        wid = lax.axis_index("s") * _NCORES + lax.axis_index("c")
        base = pl.multiple_of(jnp.minimum(wid * tpw, n - tpw), 8)
        for k in range(3):
            pltpu.sync_copy(
                m_hbm.at[pl.ds(k * n + base, tpw)], buf.at[pl.ds(k * tpw, tpw)]
            )
        zero = jnp.zeros((16,), jnp.int32)

        def chunk(c, carry):
            off = c * _LANES
            nvec = (base + off).astype(jnp.uint32) + lax.iota(jnp.uint32, 16)
            drop = _drop16(nvec)
            for k in range(3):
                sl = pl.ds(k * tpw + off, _LANES)
                buf[sl] = jnp.where(drop, zero, buf[sl])
            return carry

        lax.fori_loop(0, tpw // _LANES, chunk, 0)
        for k in range(3):
            pltpu.sync_copy(
                buf.at[pl.ds(k * tpw, tpw)], o_hbm.at[pl.ds(k * n + base, tpw)]
            )

    return body(m)


def kernel(x, edge_index, y, train_mask, test_mask, val_mask):
    n = train_mask.shape[0]
    chunk = _NTILES * _LANES
    tpw = (-(-n // chunk)) * _LANES
    t32, te32, v32 = lax.optimization_barrier(
        (
            train_mask.astype(jnp.int32),
            test_mask.astype(jnp.int32),
            val_mask.astype(jnp.int32),
        )
    )
    m = jnp.concatenate([t32, te32, v32])
    out = _node_drop_masks(m, n=n, tpw=tpw).astype(jnp.bool_)
    z = lax.optimization_barrier(jnp.int32(0))
    x_out = lax.bitcast_convert_type(
        lax.bitcast_convert_type(x, jnp.int32) ^ z, jnp.float32
    )
    e_out = edge_index ^ z.astype(edge_index.dtype)
    return (x_out, e_out, y, out[0:n], out[2 * n:3 * n], out[n:2 * n])
